# baseline (speedup 1.0000x reference)
import functools

import jax
import jax.numpy as jnp
from jax import lax
from jax.experimental import pallas as pl
from jax.experimental.pallas import tpu as pltpu
from jax.experimental.pallas import tpu_sc as plsc

N = 10000
E = 320000
D = 128
L = 14

NCORE = 2
NSUB = 16
LANES = 16
NW = NCORE * NSUB

K = 128
NCH = 80
EPT = NCH * K
EPAD = NW * EPT
NDUM = 8

RC = 64
NRC = N // RC
RREM = N - NRC * RC

_sc_mesh = plsc.VectorSubcoreMesh(core_axis_name="c", subcore_axis_name="s")


@functools.partial(
    pl.kernel,
    out_type=jax.ShapeDtypeStruct((NCORE, N, D), jnp.float32),
    mesh=_sc_mesh,
    scratch_types=[
        pltpu.VMEM((K,), jnp.int32),
        pltpu.VMEM((K,), jnp.int32),
        pltpu.VMEM((K, D), jnp.float32),
        pltpu.VMEM((K,), jnp.int32),
        pltpu.VMEM((K,), jnp.int32),
        pltpu.VMEM((K, D), jnp.float32),
        pltpu.VMEM((RC, D), jnp.float32),
        pltpu.VMEM_SHARED((N + NDUM, D), jnp.float32),
        pltpu.SemaphoreType.DMA,
        pltpu.SemaphoreType.DMA,
    ],
)
def _sc_segsum(src, dst, h, out, src_v0, dst_v0, rows0, src_v1, dst_v1,
               rows1, stage, agg, sem0, sem1):
    c = lax.axis_index("c")
    s = lax.axis_index("s")

    @pl.loop(0, RC)
    def _zrow(i):
        for k in range(D // LANES):
            stage[i, pl.ds(k * LANES, LANES)] = jnp.zeros((LANES,),
                                                          jnp.float32)

    @pl.loop(0, (NRC + NSUB - 1) // NSUB)
    def _zchunk(m):
        jj = m * NSUB + s

        @pl.when(jj < NRC)
        def _():
            pltpu.sync_copy(stage, agg.at[pl.ds(jj * RC, RC)])

    @pl.when(s == NRC % NSUB)
    def _():
        pltpu.sync_copy(stage.at[pl.ds(0, RREM)],
                        agg.at[pl.ds(NRC * RC, RREM)])

    plsc.subcore_barrier()

    ebase = (c * NSUB + s) * EPT
    bufs = ((src_v0, dst_v0, rows0, sem0), (src_v1, dst_v1, rows1, sem1))

    def _fire(j, buf):
        sv, dv, rw, sm = buf
        pltpu.sync_copy(src.at[pl.ds(ebase + j * K, K)], sv)
        pltpu.sync_copy(dst.at[pl.ds(ebase + j * K, K)], dv)
        pltpu.async_copy(h.at[sv], rw, sm)

    def _drain(buf):
        sv, dv, rw, sm = buf
        pltpu.make_async_copy(h.at[sv], rw, sm).wait()
        pltpu.sync_copy(rw, agg.at[dv], add=True)

    _fire(0, bufs[0])

    @pl.loop(0, (NCH - 2) // 2)
    def _chunk(m):
        j = 2 * m
        _fire(j + 1, bufs[1])
        _drain(bufs[0])
        _fire(j + 2, bufs[0])
        _drain(bufs[1])

    _drain(bufs[0])
    _fire(NCH - 1, bufs[1])
    _drain(bufs[1])

    plsc.subcore_barrier()

    @pl.loop(0, (NRC + NSUB - 1) // NSUB)
    def _wchunk(m):
        jj = m * NSUB + s

        @pl.when(jj < NRC)
        def _():
            pltpu.sync_copy(agg.at[pl.ds(jj * RC, RC)], stage)
            pltpu.sync_copy(stage, out.at[c, pl.ds(jj * RC, RC), :])

    @pl.when(s == NRC % NSUB)
    def _():
        pltpu.sync_copy(agg.at[pl.ds(NRC * RC, RREM)],
                        stage.at[pl.ds(0, RREM)])
        pltpu.sync_copy(stage.at[pl.ds(0, RREM)],
                        out.at[c, pl.ds(NRC * RC, RREM), :])


def _mm_body(x_ref, w_ref, h_ref):
    h_ref[...] = jnp.dot(x_ref[...], w_ref[...],
                         preferred_element_type=jnp.float32)


_mm = pl.pallas_call(
    _mm_body, out_shape=jax.ShapeDtypeStruct((N, D), jnp.float32))


def _layer_body(aggp_ref, h_ref, deg_ref, b_ref, w_ref, temp_ref, flag_ref,
                hout_ref, x_ref, tout_ref):
    agg = aggp_ref[0] + aggp_ref[1] + h_ref[...]
    deg = deg_ref[0, :, 0] + deg_ref[1, :, 0] + 1.0
    y = jnp.maximum(agg * (1.0 / deg)[:, None] + b_ref[0], 0.0)
    f = flag_ref[0, 0]
    y = y + f * temp_ref[...]
    x_ref[...] = y
    tout_ref[...] = f * y + (1.0 - f) * temp_ref[...]
    hout_ref[...] = jnp.dot(y, w_ref[...], preferred_element_type=jnp.float32)


_layer = pl.pallas_call(
    _layer_body,
    out_shape=(jax.ShapeDtypeStruct((N, D), jnp.float32),
               jax.ShapeDtypeStruct((N, D), jnp.float32),
               jax.ShapeDtypeStruct((N, D), jnp.float32)))


def kernel(mesh, shape_features, W, b):
    srcp = jnp.concatenate([mesh[0], jnp.zeros((EPAD - E,), jnp.int32)])
    dstp = jnp.concatenate([mesh[1], jnp.full((EPAD - E,), N, jnp.int32)])

    deg2 = _sc_segsum(srcp, dstp, jnp.ones((N, D), jnp.float32))

    x = shape_features
    h = _mm(x, W[0])
    w_next = jnp.concatenate([W[1:], W[:1]])
    flags = jnp.tile(jnp.array([0.0, 1.0], jnp.float32), L // 2)

    def step(carry, xs):
        h, temp, _ = carry
        w_i, b_i, f_i = xs
        aggp = _sc_segsum(srcp, dstp, h)
        h, x, temp = _layer(aggp, h, deg2, b_i[None], w_i, temp,
                            f_i[None, None])
        return (h, temp, x), None

    (_, _, x), _ = lax.scan(step, (h, x, x), (w_next, b, flags))
    return x


# speedup vs baseline: 3.6696x; 3.6696x over previous
import functools

import jax
import jax.numpy as jnp
from jax import lax
from jax.experimental import pallas as pl
from jax.experimental.pallas import tpu as pltpu
from jax.experimental.pallas import tpu_sc as plsc

N = 10000
E = 320000
D = 128
L = 14

NCORE = 2
NSUB = 16
LANES = 16
NW = NCORE * NSUB

EPT = E // NW
K = 128
NCH = EPT // K
REM = EPT - NCH * K

RC = 64
NRC = N // RC
RREM = N - NRC * RC

_sc_mesh = plsc.VectorSubcoreMesh(core_axis_name="c", subcore_axis_name="s")


@functools.partial(
    pl.kernel,
    out_type=jax.ShapeDtypeStruct((NCORE, N, D), jnp.float32),
    mesh=_sc_mesh,
    scratch_types=[
        pltpu.VMEM((K,), jnp.int32),
        pltpu.VMEM((K,), jnp.int32),
        pltpu.VMEM((K, D), jnp.float32),
        pltpu.VMEM((K,), jnp.int32),
        pltpu.VMEM((K,), jnp.int32),
        pltpu.VMEM((K, D), jnp.float32),
        pltpu.VMEM((REM,), jnp.int32),
        pltpu.VMEM((REM,), jnp.int32),
        pltpu.VMEM((REM, D), jnp.float32),
        pltpu.VMEM((RC, D), jnp.float32),
        pltpu.VMEM_SHARED((N, D), jnp.float32),
        pltpu.SemaphoreType.DMA,
        pltpu.SemaphoreType.DMA,
        pltpu.SemaphoreType.DMA,
        pltpu.SemaphoreType.DMA,
        pltpu.SemaphoreType.DMA,
        pltpu.SemaphoreType.DMA,
    ],
)
def _sc_segsum(src, dst, h, out, src_v0, dst_v0, rows0, src_v1, dst_v1,
               rows1, srcr_v, dstr_v, rowsr, stage, agg, sem0, sem1,
               isems0, isemd0, isems1, isemd1):
    c = lax.axis_index("c")
    s = lax.axis_index("s")

    @pl.loop(0, RC)
    def _zrow(i):
        for k in range(D // LANES):
            stage[i, pl.ds(k * LANES, LANES)] = jnp.zeros((LANES,),
                                                          jnp.float32)

    @pl.loop(0, (NRC + NSUB - 1) // NSUB)
    def _zchunk(m):
        jj = m * NSUB + s

        @pl.when(jj < NRC)
        def _():
            pltpu.sync_copy(stage, agg.at[pl.ds(jj * RC, RC)])

    @pl.when(s == NRC % NSUB)
    def _():
        pltpu.sync_copy(stage.at[pl.ds(0, RREM)],
                        agg.at[pl.ds(NRC * RC, RREM)])

    plsc.subcore_barrier()

    ebase = (c * NSUB + s) * EPT
    bufs = ((src_v0, dst_v0, rows0, sem0, isems0, isemd0),
            (src_v1, dst_v1, rows1, sem1, isems1, isemd1))

    def _fire_idx(j, buf):
        sv, dv, rw, sm, isms, ismd = buf
        pltpu.async_copy(src.at[pl.ds(ebase + j * K, K)], sv, isms)
        pltpu.async_copy(dst.at[pl.ds(ebase + j * K, K)], dv, ismd)

    def _wait_idx(j, buf):
        sv, dv, rw, sm, isms, ismd = buf
        pltpu.make_async_copy(src.at[pl.ds(ebase + j * K, K)], sv,
                              isms).wait()
        pltpu.make_async_copy(dst.at[pl.ds(ebase + j * K, K)], dv,
                              ismd).wait()

    def _fire_gather(buf):
        sv, dv, rw, sm, isms, ismd = buf
        pltpu.async_copy(h.at[sv], rw, sm)

    def _scatter(buf):
        sv, dv, rw, sm, isms, ismd = buf
        pltpu.make_async_copy(h.at[sv], rw, sm).wait()
        pltpu.sync_copy(rw, agg.at[dv], add=True)

    _fire_idx(0, bufs[0])
    _wait_idx(0, bufs[0])
    _fire_gather(bufs[0])
    _fire_idx(1, bufs[1])

    @pl.loop(0, NCH // 2 - 1)
    def _chunk(m):
        j = 2 * m
        _wait_idx(j + 1, bufs[1])
        _fire_gather(bufs[1])
        _scatter(bufs[0])
        _fire_idx(j + 2, bufs[0])
        _wait_idx(j + 2, bufs[0])
        _fire_gather(bufs[0])
        _scatter(bufs[1])
        _fire_idx(j + 3, bufs[1])

    _wait_idx(NCH - 1, bufs[1])
    _fire_gather(bufs[1])
    _scatter(bufs[0])
    _scatter(bufs[1])

    base = ebase + NCH * K
    pltpu.sync_copy(src.at[pl.ds(base, REM)], srcr_v)
    pltpu.sync_copy(dst.at[pl.ds(base, REM)], dstr_v)
    pltpu.async_copy(h.at[srcr_v], rowsr, sem0).wait()
    pltpu.sync_copy(rowsr, agg.at[dstr_v], add=True)

    plsc.subcore_barrier()

    @pl.loop(0, (NRC + NSUB - 1) // NSUB)
    def _wchunk(m):
        jj = m * NSUB + s

        @pl.when(jj < NRC)
        def _():
            pltpu.sync_copy(agg.at[pl.ds(jj * RC, RC)], stage)
            pltpu.sync_copy(stage, out.at[c, pl.ds(jj * RC, RC), :])

    @pl.when(s == NRC % NSUB)
    def _():
        pltpu.sync_copy(agg.at[pl.ds(NRC * RC, RREM)],
                        stage.at[pl.ds(0, RREM)])
        pltpu.sync_copy(stage.at[pl.ds(0, RREM)],
                        out.at[c, pl.ds(NRC * RC, RREM), :])


def _mm_body(x_ref, w_ref, h_ref):
    h_ref[...] = jnp.dot(x_ref[...], w_ref[...],
                         preferred_element_type=jnp.float32)


_mm = pl.pallas_call(
    _mm_body, out_shape=jax.ShapeDtypeStruct((N, D), jnp.float32))


def _layer_body(aggp_ref, h_ref, deg_ref, b_ref, w_ref, temp_ref, flag_ref,
                hout_ref, x_ref, tout_ref):
    agg = aggp_ref[0] + aggp_ref[1] + h_ref[...]
    deg = deg_ref[0, :, 0] + deg_ref[1, :, 0] + 1.0
    y = jnp.maximum(agg * (1.0 / deg)[:, None] + b_ref[0], 0.0)
    f = flag_ref[0, 0]
    y = y + f * temp_ref[...]
    x_ref[...] = y
    tout_ref[...] = f * y + (1.0 - f) * temp_ref[...]
    hout_ref[...] = jnp.dot(y, w_ref[...], preferred_element_type=jnp.float32)


_layer = pl.pallas_call(
    _layer_body,
    out_shape=(jax.ShapeDtypeStruct((N, D), jnp.float32),
               jax.ShapeDtypeStruct((N, D), jnp.float32),
               jax.ShapeDtypeStruct((N, D), jnp.float32)))


def kernel(mesh, shape_features, W, b):
    src = mesh[0]
    dst = mesh[1]

    deg2 = _sc_segsum(src, dst, jnp.ones((N, D), jnp.float32))

    x = shape_features
    h = _mm(x, W[0])
    w_next = jnp.concatenate([W[1:], W[:1]])
    flags = jnp.tile(jnp.array([0.0, 1.0], jnp.float32), L // 2)

    def step(carry, xs):
        h, temp, _ = carry
        w_i, b_i, f_i = xs
        aggp = _sc_segsum(src, dst, h)
        h, x, temp = _layer(aggp, h, deg2, b_i[None], w_i, temp,
                            f_i[None, None])
        return (h, temp, x), None

    (_, _, x), _ = lax.scan(step, (h, x, x), (w_next, b, flags))
    return x


# speedup vs baseline: 3.8175x; 1.0403x over previous
import functools

import jax
import jax.numpy as jnp
from jax import lax
from jax.experimental import pallas as pl
from jax.experimental.pallas import tpu as pltpu
from jax.experimental.pallas import tpu_sc as plsc

N = 10000
E = 320000
D = 128
L = 14

NCORE = 2
NSUB = 16
LANES = 16
NW = NCORE * NSUB

EPT = E // NW
K = 128
NCH = EPT // K
REM = EPT - NCH * K

RC = 64
NRC = N // RC
RREM = N - NRC * RC

_sc_mesh = plsc.VectorSubcoreMesh(core_axis_name="c", subcore_axis_name="s")


@functools.partial(
    pl.kernel,
    out_type=jax.ShapeDtypeStruct((NCORE, N, D), jnp.float32),
    mesh=_sc_mesh,
    scratch_types=[
        pltpu.VMEM((K,), jnp.int32),
        pltpu.VMEM((K,), jnp.int32),
        pltpu.VMEM((K, D), jnp.float32),
        pltpu.VMEM((K,), jnp.int32),
        pltpu.VMEM((K,), jnp.int32),
        pltpu.VMEM((K, D), jnp.float32),
        pltpu.VMEM((REM,), jnp.int32),
        pltpu.VMEM((REM,), jnp.int32),
        pltpu.VMEM((REM, D), jnp.float32),
        pltpu.VMEM((RC, D), jnp.float32),
        pltpu.VMEM_SHARED((N, D), jnp.float32),
        pltpu.SemaphoreType.DMA,
        pltpu.SemaphoreType.DMA,
        pltpu.SemaphoreType.DMA,
        pltpu.SemaphoreType.DMA,
        pltpu.SemaphoreType.DMA,
        pltpu.SemaphoreType.DMA,
    ],
)
def _sc_segsum(src, dst, h, out, src_v0, dst_v0, rows0, src_v1, dst_v1,
               rows1, srcr_v, dstr_v, rowsr, stage, agg, sem0, sem1,
               isems0, isemd0, isems1, isemd1):
    c = lax.axis_index("c")
    s = lax.axis_index("s")

    @pl.loop(0, RC)
    def _zrow(i):
        for k in range(D // LANES):
            stage[i, pl.ds(k * LANES, LANES)] = jnp.zeros((LANES,),
                                                          jnp.float32)

    @pl.loop(0, (NRC + NSUB - 1) // NSUB)
    def _zchunk(m):
        jj = m * NSUB + s

        @pl.when(jj < NRC)
        def _():
            pltpu.sync_copy(stage, agg.at[pl.ds(jj * RC, RC)])

    @pl.when(s == NRC % NSUB)
    def _():
        pltpu.sync_copy(stage.at[pl.ds(0, RREM)],
                        agg.at[pl.ds(NRC * RC, RREM)])

    plsc.subcore_barrier()

    ebase = (c * NSUB + s) * EPT
    bufs = ((src_v0, dst_v0, rows0, sem0, isems0, isemd0),
            (src_v1, dst_v1, rows1, sem1, isems1, isemd1))

    def _fire_idx(j, buf):
        sv, dv, rw, sm, isms, ismd = buf
        pltpu.async_copy(src.at[pl.ds(ebase + j * K, K)], sv, isms)
        pltpu.async_copy(dst.at[pl.ds(ebase + j * K, K)], dv, ismd)

    def _wait_idx(j, buf):
        sv, dv, rw, sm, isms, ismd = buf
        pltpu.make_async_copy(src.at[pl.ds(ebase + j * K, K)], sv,
                              isms).wait()
        pltpu.make_async_copy(dst.at[pl.ds(ebase + j * K, K)], dv,
                              ismd).wait()

    def _fire_gather(buf):
        sv, dv, rw, sm, isms, ismd = buf
        pltpu.async_copy(h.at[sv], rw, sm)

    def _scatter(buf):
        sv, dv, rw, sm, isms, ismd = buf
        pltpu.make_async_copy(h.at[sv], rw, sm).wait()
        pltpu.sync_copy(rw, agg.at[dv], add=True)

    _fire_idx(0, bufs[0])
    _wait_idx(0, bufs[0])
    _fire_gather(bufs[0])
    _fire_idx(1, bufs[1])

    @pl.loop(0, NCH // 2 - 1)
    def _chunk(m):
        j = 2 * m
        _wait_idx(j + 1, bufs[1])
        _fire_gather(bufs[1])
        _scatter(bufs[0])
        _fire_idx(j + 2, bufs[0])
        _wait_idx(j + 2, bufs[0])
        _fire_gather(bufs[0])
        _scatter(bufs[1])
        _fire_idx(j + 3, bufs[1])

    _wait_idx(NCH - 1, bufs[1])
    _fire_gather(bufs[1])
    _scatter(bufs[0])
    _scatter(bufs[1])

    base = ebase + NCH * K
    pltpu.sync_copy(src.at[pl.ds(base, REM)], srcr_v)
    pltpu.sync_copy(dst.at[pl.ds(base, REM)], dstr_v)
    pltpu.async_copy(h.at[srcr_v], rowsr, sem0).wait()
    pltpu.sync_copy(rowsr, agg.at[dstr_v], add=True)

    plsc.subcore_barrier()

    @pl.loop(0, (NRC + NSUB - 1) // NSUB)
    def _wchunk(m):
        jj = m * NSUB + s

        @pl.when(jj < NRC)
        def _():
            pltpu.sync_copy(agg.at[pl.ds(jj * RC, RC)], stage)
            pltpu.sync_copy(stage, out.at[c, pl.ds(jj * RC, RC), :])

    @pl.when(s == NRC % NSUB)
    def _():
        pltpu.sync_copy(agg.at[pl.ds(NRC * RC, RREM)],
                        stage.at[pl.ds(0, RREM)])
        pltpu.sync_copy(stage.at[pl.ds(0, RREM)],
                        out.at[c, pl.ds(NRC * RC, RREM), :])


def _mm_body(x_ref, w_ref, h_ref):
    h_ref[...] = jnp.dot(x_ref[...], w_ref[...],
                         preferred_element_type=jnp.float32)


_mm = pl.pallas_call(
    _mm_body, out_shape=jax.ShapeDtypeStruct((N, D), jnp.float32))


def _epilogue(aggp_ref, h_ref, deg_ref, b_ref):
    agg = aggp_ref[0] + aggp_ref[1] + h_ref[...]
    deg = deg_ref[0, :, 0] + deg_ref[1, :, 0] + 1.0
    return jnp.maximum(agg * (1.0 / deg)[:, None] + b_ref[0], 0.0)


def _layer_a_body(aggp_ref, h_ref, deg_ref, b_ref, w_ref, hout_ref):
    y = _epilogue(aggp_ref, h_ref, deg_ref, b_ref)
    hout_ref[...] = jnp.dot(y, w_ref[...], preferred_element_type=jnp.float32)


_layer_a = pl.pallas_call(
    _layer_a_body, out_shape=jax.ShapeDtypeStruct((N, D), jnp.float32))


def _layer_b_body(aggp_ref, h_ref, deg_ref, b_ref, w_ref, temp_ref,
                  hout_ref, x_ref):
    y = _epilogue(aggp_ref, h_ref, deg_ref, b_ref) + temp_ref[...]
    x_ref[...] = y
    hout_ref[...] = jnp.dot(y, w_ref[...], preferred_element_type=jnp.float32)


_layer_b = pl.pallas_call(
    _layer_b_body,
    out_shape=(jax.ShapeDtypeStruct((N, D), jnp.float32),
               jax.ShapeDtypeStruct((N, D), jnp.float32)))


def kernel(mesh, shape_features, W, b):
    src = mesh[0]
    dst = mesh[1]

    deg2 = _sc_segsum(src, dst, jnp.ones((N, D), jnp.float32))

    x = shape_features
    h = _mm(x, W[0])
    w_odd = W[1::2]
    w_next = jnp.concatenate([W[2::2], W[:1]])
    b_even = b[0::2]
    b_odd = b[1::2]

    def step(carry, xs):
        h, temp = carry
        w_o, w_n, b_e, b_o = xs
        aggp = _sc_segsum(src, dst, h)
        h = _layer_a(aggp, h, deg2, b_e[None], w_o)
        aggp = _sc_segsum(src, dst, h)
        h, temp = _layer_b(aggp, h, deg2, b_o[None], w_n, temp)
        return (h, temp), None

    (_, x), _ = lax.scan(step, (h, x), (w_odd, w_next, b_even, b_odd))
    return x


# speedup vs baseline: 3.8296x; 1.0032x over previous
import functools

import jax
import jax.numpy as jnp
from jax import lax
from jax.experimental import pallas as pl
from jax.experimental.pallas import tpu as pltpu
from jax.experimental.pallas import tpu_sc as plsc

N = 10000
E = 320000
D = 128
L = 14

NCORE = 2
NSUB = 16
LANES = 16
NW = NCORE * NSUB

EPT = E // NW
K = 128
NCH = EPT // K
REM = EPT - NCH * K

RC = 64
NRC = N // RC
RREM = N - NRC * RC

_sc_mesh = plsc.VectorSubcoreMesh(core_axis_name="c", subcore_axis_name="s")


@functools.partial(
    pl.kernel,
    out_type=jax.ShapeDtypeStruct((NCORE, N, D), jnp.float32),
    mesh=_sc_mesh,
    scratch_types=[
        pltpu.VMEM((K,), jnp.int32),
        pltpu.VMEM((K,), jnp.int32),
        pltpu.VMEM((K, D), jnp.float32),
        pltpu.VMEM((K,), jnp.int32),
        pltpu.VMEM((K,), jnp.int32),
        pltpu.VMEM((K, D), jnp.float32),
        pltpu.VMEM((REM,), jnp.int32),
        pltpu.VMEM((REM,), jnp.int32),
        pltpu.VMEM((REM, D), jnp.float32),
        pltpu.VMEM((RC, D), jnp.float32),
        pltpu.VMEM_SHARED((N, D), jnp.float32),
        pltpu.SemaphoreType.DMA,
        pltpu.SemaphoreType.DMA,
        pltpu.SemaphoreType.DMA,
        pltpu.SemaphoreType.DMA,
        pltpu.SemaphoreType.DMA,
        pltpu.SemaphoreType.DMA,
    ],
)
def _sc_segsum(src, dst, h, out, src_v0, dst_v0, rows0, src_v1, dst_v1,
               rows1, srcr_v, dstr_v, rowsr, stage, agg, sem0, sem1,
               isems0, isemd0, isems1, isemd1):
    c = lax.axis_index("c")
    s = lax.axis_index("s")

    @pl.loop(0, RC)
    def _zrow(i):
        for k in range(D // LANES):
            stage[i, pl.ds(k * LANES, LANES)] = jnp.zeros((LANES,),
                                                          jnp.float32)

    @pl.loop(0, (NRC + NSUB - 1) // NSUB)
    def _zchunk(m):
        jj = m * NSUB + s

        @pl.when(jj < NRC)
        def _():
            pltpu.sync_copy(stage, agg.at[pl.ds(jj * RC, RC)])

    @pl.when(s == NRC % NSUB)
    def _():
        pltpu.sync_copy(stage.at[pl.ds(0, RREM)],
                        agg.at[pl.ds(NRC * RC, RREM)])

    plsc.subcore_barrier()

    ebase = (c * NSUB + s) * EPT
    bufs = ((src_v0, dst_v0, rows0, sem0, isems0, isemd0),
            (src_v1, dst_v1, rows1, sem1, isems1, isemd1))

    def _fire_idx(j, buf):
        sv, dv, rw, sm, isms, ismd = buf
        pltpu.async_copy(src.at[pl.ds(ebase + j * K, K)], sv, isms)
        pltpu.async_copy(dst.at[pl.ds(ebase + j * K, K)], dv, ismd)

    def _wait_idx(j, buf):
        sv, dv, rw, sm, isms, ismd = buf
        pltpu.make_async_copy(src.at[pl.ds(ebase + j * K, K)], sv,
                              isms).wait()
        pltpu.make_async_copy(dst.at[pl.ds(ebase + j * K, K)], dv,
                              ismd).wait()

    def _fire_gather(buf):
        sv, dv, rw, sm, isms, ismd = buf
        pltpu.async_copy(h.at[sv], rw, sm)

    def _scatter(buf):
        sv, dv, rw, sm, isms, ismd = buf
        pltpu.make_async_copy(h.at[sv], rw, sm).wait()
        pltpu.sync_copy(rw, agg.at[dv], add=True)

    _fire_idx(0, bufs[0])
    _wait_idx(0, bufs[0])
    _fire_gather(bufs[0])
    _fire_idx(1, bufs[1])

    @pl.loop(0, NCH // 2 - 1)
    def _chunk(m):
        j = 2 * m
        _wait_idx(j + 1, bufs[1])
        _fire_gather(bufs[1])
        _scatter(bufs[0])
        _fire_idx(j + 2, bufs[0])
        _wait_idx(j + 2, bufs[0])
        _fire_gather(bufs[0])
        _scatter(bufs[1])
        _fire_idx(j + 3, bufs[1])

    _wait_idx(NCH - 1, bufs[1])
    _fire_gather(bufs[1])
    _scatter(bufs[0])
    _scatter(bufs[1])

    base = ebase + NCH * K
    pltpu.sync_copy(src.at[pl.ds(base, REM)], srcr_v)
    pltpu.sync_copy(dst.at[pl.ds(base, REM)], dstr_v)
    pltpu.async_copy(h.at[srcr_v], rowsr, sem0).wait()
    pltpu.sync_copy(rowsr, agg.at[dstr_v], add=True)

    plsc.subcore_barrier()

    @pl.loop(0, (NRC + NSUB - 1) // NSUB)
    def _wchunk(m):
        jj = m * NSUB + s

        @pl.when(jj < NRC)
        def _():
            pltpu.sync_copy(agg.at[pl.ds(jj * RC, RC)], stage)
            pltpu.sync_copy(stage, out.at[c, pl.ds(jj * RC, RC), :])

    @pl.when(s == NRC % NSUB)
    def _():
        pltpu.sync_copy(agg.at[pl.ds(NRC * RC, RREM)],
                        stage.at[pl.ds(0, RREM)])
        pltpu.sync_copy(stage.at[pl.ds(0, RREM)],
                        out.at[c, pl.ds(NRC * RC, RREM), :])


RT = 2000

_bs2 = pl.BlockSpec((RT, D), lambda r: (r, 0))
_bs3 = pl.BlockSpec((NCORE, RT, D), lambda r: (0, r, 0))
_bsw = pl.BlockSpec((D, D), lambda r: (0, 0))
_bsb = pl.BlockSpec((1, D), lambda r: (0, 0))
_out2 = jax.ShapeDtypeStruct((N, D), jnp.float32)


def _mm_body(x_ref, w_ref, h_ref):
    h_ref[...] = jnp.dot(x_ref[...], w_ref[...],
                         preferred_element_type=jnp.float32)


_mm = pl.pallas_call(
    _mm_body, grid=(N // RT,), in_specs=[_bs2, _bsw], out_specs=_bs2,
    out_shape=_out2)


def _epilogue(aggp_ref, h_ref, deg_ref, b_ref):
    agg = aggp_ref[0] + aggp_ref[1] + h_ref[...]
    deg = deg_ref[0, :, 0] + deg_ref[1, :, 0] + 1.0
    return jnp.maximum(agg * (1.0 / deg)[:, None] + b_ref[0], 0.0)


def _layer_a_body(aggp_ref, h_ref, deg_ref, b_ref, w_ref, hout_ref):
    y = _epilogue(aggp_ref, h_ref, deg_ref, b_ref)
    hout_ref[...] = jnp.dot(y, w_ref[...], preferred_element_type=jnp.float32)


_layer_a = pl.pallas_call(
    _layer_a_body, grid=(N // RT,),
    in_specs=[_bs3, _bs2, _bs3, _bsb, _bsw], out_specs=_bs2,
    out_shape=_out2)


def _layer_b_body(aggp_ref, h_ref, deg_ref, b_ref, w_ref, temp_ref,
                  hout_ref, x_ref):
    y = _epilogue(aggp_ref, h_ref, deg_ref, b_ref) + temp_ref[...]
    x_ref[...] = y
    hout_ref[...] = jnp.dot(y, w_ref[...], preferred_element_type=jnp.float32)


_layer_b = pl.pallas_call(
    _layer_b_body, grid=(N // RT,),
    in_specs=[_bs3, _bs2, _bs3, _bsb, _bsw, _bs2],
    out_specs=(_bs2, _bs2), out_shape=(_out2, _out2))


def kernel(mesh, shape_features, W, b):
    src = mesh[0]
    dst = mesh[1]

    deg2 = _sc_segsum(src, dst, jnp.ones((N, D), jnp.float32))

    x = shape_features
    h = _mm(x, W[0])
    w_odd = W[1::2]
    w_next = jnp.concatenate([W[2::2], W[:1]])
    b_even = b[0::2]
    b_odd = b[1::2]

    def step(carry, xs):
        h, temp = carry
        w_o, w_n, b_e, b_o = xs
        aggp = _sc_segsum(src, dst, h)
        h = _layer_a(aggp, h, deg2, b_e[None], w_o)
        aggp = _sc_segsum(src, dst, h)
        h, temp = _layer_b(aggp, h, deg2, b_o[None], w_n, temp)
        return (h, temp), None

    (_, x), _ = lax.scan(step, (h, x), (w_odd, w_next, b_even, b_odd))
    return x
